# Initial kernel scaffold; baseline (speedup 1.0000x reference)
import jax
import jax.numpy as jnp
from jax.experimental import pallas as pl


def kernel(x, W1, b1, W2, b2, W3, b3, Wp, bp):
    raise NotImplementedError("write your pallas kernel here")


# speedup vs baseline: 6.6157x; 6.6157x over previous
import functools

import jax
import jax.numpy as jnp
from jax.experimental import pallas as pl

KNN = 16
RB = 256


def _leaky(h):
    return jnp.where(h >= 0, h, 0.01 * h)


def _neg_dist(xb, xa):
    inner = jax.lax.dot_general(
        xb, xa, (((1,), (1,)), ((), ())),
        preferred_element_type=jnp.float32)
    sqb = jnp.sum(xb * xb, axis=1, keepdims=True)
    ones = jnp.ones((1, xa.shape[1]), dtype=jnp.float32)
    sqa = jax.lax.dot_general(
        ones, xa * xa, (((1,), (1,)), ((), ())),
        precision=jax.lax.Precision.HIGHEST,
        preferred_element_type=jnp.float32)
    return 2.0 * inner - sqb - sqa


def _three_plane(x):
    p1 = x.astype(jnp.bfloat16)
    r1 = x - p1.astype(jnp.float32)
    p2 = r1.astype(jnp.bfloat16)
    p3 = (r1 - p2.astype(jnp.float32)).astype(jnp.bfloat16)
    return jnp.concatenate([p1, p2, p3], axis=1)


def _edge_core(xb, xa, w, b, n, c):
    rb = xb.shape[0]
    cout = w.shape[1]
    nd = _neg_dist(xb, xa)
    xa3 = _three_plane(xa)
    iota = jax.lax.broadcasted_iota(jnp.int32, (rb, n), 1)
    macc0 = jnp.full((rb, cout), -jnp.inf, dtype=jnp.float32)
    neginf = jnp.float32(-jnp.inf)

    def body(_, carry):
        nd, macc = carry
        m = jnp.max(nd, axis=1, keepdims=True)
        cand = jnp.where(nd == m, iota, n)
        idx = jnp.min(cand, axis=1, keepdims=True)
        onehot = iota == idx
        g = jax.lax.dot_general(
            onehot.astype(jnp.bfloat16), xa3, (((1,), (0,)), ((), ())),
            preferred_element_type=jnp.float32)
        xj = g[:, :c] + g[:, c:2 * c] + g[:, 2 * c:]
        e = jnp.concatenate([xb, xj - xb], axis=1)
        h = jax.lax.dot_general(
            e, w, (((1,), (0,)), ((), ())),
            preferred_element_type=jnp.float32) + b
        macc = jnp.maximum(macc, h)
        nd = jnp.where(onehot, neginf, nd)
        return nd, macc

    _, macc = jax.lax.fori_loop(0, KNN, body, (nd, macc0))
    return macc


def _edge_kernel(x_blk_ref, x_all_ref, w_ref, b_ref, out_ref, *, n, c):
    macc = _edge_core(x_blk_ref[0], x_all_ref[0], w_ref[...], b_ref[0], n, c)
    out_ref[0] = _leaky(macc)


def _edge_proj_kernel(x_blk_ref, x_all_ref, w_ref, b_ref,
                      x0_ref, x1_ref, wp_ref, bp_ref, out_ref, *, n, c, c0):
    xb = x_blk_ref[0]
    x3 = _leaky(_edge_core(xb, x_all_ref[0], w_ref[...], b_ref[0], n, c))
    wp0 = wp_ref[:c0, :]
    wp1 = wp_ref[c0:c0 + 64, :]
    wp2 = wp_ref[c0 + 64:c0 + 128, :]
    wp3 = wp_ref[c0 + 128:, :]
    acc = jnp.dot(x0_ref[0], wp0, preferred_element_type=jnp.float32)
    acc += jnp.dot(x1_ref[0], wp1, preferred_element_type=jnp.float32)
    acc += jnp.dot(xb, wp2, preferred_element_type=jnp.float32)
    acc += jnp.dot(x3, wp3, preferred_element_type=jnp.float32)
    out_ref[0] = acc + bp_ref[0]


def _edge_conv(x, W, b, interpret=False):
    B, N, C = x.shape
    cout = W.shape[1]
    grid = (B, N // RB)
    return pl.pallas_call(
        functools.partial(_edge_kernel, n=N, c=C),
        grid=grid,
        in_specs=[
            pl.BlockSpec((1, RB, C), lambda bb, ii: (bb, ii, 0)),
            pl.BlockSpec((1, N, C), lambda bb, ii: (bb, 0, 0)),
            pl.BlockSpec((2 * C, cout), lambda bb, ii: (0, 0)),
            pl.BlockSpec((1, cout), lambda bb, ii: (0, 0)),
        ],
        out_specs=pl.BlockSpec((1, RB, cout), lambda bb, ii: (bb, ii, 0)),
        out_shape=jax.ShapeDtypeStruct((B, N, cout), jnp.float32),
        interpret=interpret,
    )(x, x, W, b.reshape(1, cout))


def _edge_conv_proj(x2, W, b, x0, x1, Wp, bp, interpret=False):
    B, N, C = x2.shape
    cout = W.shape[1]
    c0 = x0.shape[2]
    grid = (B, N // RB)
    return pl.pallas_call(
        functools.partial(_edge_proj_kernel, n=N, c=C, c0=c0),
        grid=grid,
        in_specs=[
            pl.BlockSpec((1, RB, C), lambda bb, ii: (bb, ii, 0)),
            pl.BlockSpec((1, N, C), lambda bb, ii: (bb, 0, 0)),
            pl.BlockSpec((2 * C, cout), lambda bb, ii: (0, 0)),
            pl.BlockSpec((1, cout), lambda bb, ii: (0, 0)),
            pl.BlockSpec((1, RB, c0), lambda bb, ii: (bb, ii, 0)),
            pl.BlockSpec((1, RB, 64), lambda bb, ii: (bb, ii, 0)),
            pl.BlockSpec((c0 + 192, 64), lambda bb, ii: (0, 0)),
            pl.BlockSpec((1, 64), lambda bb, ii: (0, 0)),
        ],
        out_specs=pl.BlockSpec((1, RB, 64), lambda bb, ii: (bb, ii, 0)),
        out_shape=jax.ShapeDtypeStruct((B, N, 64), jnp.float32),
        interpret=interpret,
    )(x2, x2, W, b.reshape(1, cout), x0, x1, Wp, bp.reshape(1, 64))


def kernel(x, W1, b1, W2, b2, W3, b3, Wp, bp, interpret=False):
    x1 = _edge_conv(x, W1, b1, interpret)
    x2 = _edge_conv(x1, W2, b2, interpret)
    return _edge_conv_proj(x2, W3, b3, x, x1, Wp, bp, interpret)
